# baseline (speedup 1.0000x reference)
import jax
import jax.numpy as jnp
from jax import lax
from jax.experimental import pallas as pl
from jax.experimental.pallas import tpu as pltpu
from jax.experimental.pallas import tpu_sc as plsc

NC, NS = 2, 16
NW = NC * NS
BT = 4096
TT = 200
D = 32
GROUP = 128
NCHUNK = BT * TT // GROUP
GPW = NCHUNK // NW
NBUF = 5


def _emb_body(idx_hbm, w_hbm, out_hbm, idx_v, g_v, gt_v, gsem, *osems):
    wid = lax.axis_index("s") * NC + lax.axis_index("c")
    r0 = wid * GPW
    pltpu.sync_copy(idx_hbm.at[pl.ds(r0, GPW)], idx_v)
    iota = lax.iota(jnp.int32, 16)
    colidx = [(iota + k) & 15 for k in range(16)]
    sidx = [c * GROUP + iota for c in colidx]

    def fire_g(k, b):
        pltpu.async_copy(w_hbm.at[idx_v.at[k]], g_v.at[b], gsem)

    def drain_g(b):
        pltpu.make_async_copy(w_hbm.at[pl.ds(0, GROUP)], g_v.at[b], gsem).wait()

    def transpose(b):
        @plsc.parallel_loop(0, GROUP, step=16)
        def _blk(j0):
            rows = iota + j0
            for d0 in (0, 16):
                for k in range(16):
                    vals = plsc.load_gather(g_v.at[b], [rows, colidx[k] + d0])
                    plsc.store_scatter(
                        gt_v.at[b], [sidx[k] + (d0 * GROUP + j0)], vals
                    )

    def fire_out(k, b):
        r = r0 + k
        t = (r >> 8) * 8 + (r & 7)
        b_hi = (r >> 3) & 31
        for d_hi in range(4):
            pltpu.async_copy(
                gt_v.at[b].at[pl.ds(d_hi * 1024, 1024)],
                out_hbm.at[t, d_hi, b_hi],
                osems[b],
            )

    def drain_out(b):
        for d_hi in range(4):
            pltpu.make_async_copy(
                gt_v.at[b].at[pl.ds(d_hi * 1024, 1024)],
                out_hbm.at[0, d_hi, 0],
                osems[b],
            ).wait()

    for b in range(NBUF):
        fire_g(b, b)

    @pl.loop(0, GPW - NBUF, step=NBUF)
    def _steady(kk):
        for b in range(NBUF):
            k = kk + b
            drain_g(b)
            transpose(b)
            fire_out(k, b)
            drain_out(b)
            fire_g(k + NBUF, b)

    for b in range(NBUF):
        k = GPW - NBUF + b
        drain_g(b)
        transpose(b)
        fire_out(k, b)
    for b in range(NBUF):
        drain_out(b)


VB = 7812
BPW = 245
BIMAX = VB - 1
NB2 = 7
TAIL0 = VB * 128


def _tr_body(wt_hbm, w2t_hbm, wl_hbm, s3, *rest):
    gts = rest[:NB2]
    gsem = rest[NB2]
    osems = rest[NB2 + 1:]
    wid = lax.axis_index("s") * NC + lax.axis_index("c")
    base = wid * BPW
    iota = lax.iota(jnp.int32, 16)
    colidx = [(iota + k) & 15 for k in range(16)]
    sidxa = [c * D + iota for c in colidx]
    dhi = [iota >> 3, (iota >> 3) + 2]
    dlo = iota & 7

    def c0_of(n):
        return pl.multiple_of(jnp.minimum(base + n, BIMAX) * 128, 128)

    def fire_in(n, s):
        c0 = c0_of(n)
        for r in range(4):
            pltpu.async_copy(
                wt_hbm.at[pl.ds(8 * r, 8), pl.ds(c0, 128)],
                s3.at[s * 4 + r],
                gsem,
            )

    def drain_in(s):
        for r in range(4):
            pltpu.make_async_copy(
                wt_hbm.at[pl.ds(0, 8), pl.ds(0, 128)], s3.at[s * 4 + r], gsem
            ).wait()

    def transpose(s):
        @plsc.parallel_loop(0, 8)
        def _blk(j0i):
            j0 = j0i * 16
            for h in (0, 1):
                for k in range(16):
                    vals = plsc.load_gather(
                        s3.at[pl.ds(s * 4, 4)], [dhi[h], dlo, colidx[k] + j0]
                    )
                    plsc.store_scatter(
                        gts[s], [sidxa[k] + (j0 * D + 16 * h)], vals
                    )

    def fire_out(n, s):
        pltpu.async_copy(
            gts[s],
            wl_hbm.at[pl.ds(pl.multiple_of(c0_of(n) * D, 4096), 4096)],
            osems[s],
        )

    def drain_out(s):
        pltpu.make_async_copy(
            gts[s], wl_hbm.at[pl.ds(0, 4096)], osems[s]
        ).wait()

    for s in range(NB2):
        fire_in(s, s)

    @pl.loop(0, BPW - NB2, step=NB2)
    def _steady(nn):
        for s in range(NB2):
            n = nn + s
            drain_in(s)
            transpose(s)
            fire_out(n, s)
            drain_out(s)
            fire_in(n + NB2, s)

    for s in range(NB2):
        n = BPW - NB2 + s
        drain_in(s)
        transpose(s)
        fire_out(n, s)
    for s in range(NB2):
        drain_out(s)

    @pl.when(wid == NW - 1)
    def _tail():
        for r in range(4):
            pltpu.sync_copy(w2t_hbm.at[pl.ds(8 * r, 8)], s3.at[r])
        transpose(0)
        pltpu.sync_copy(
            gts[0].at[pl.ds(0, 64 * D)], wl_hbm.at[pl.ds(TAIL0 * D, 64 * D)]
        )


def _relayout_table(W):
    w2t = jnp.pad(W[TAIL0:], ((0, 64), (0, 0))).T
    wl = pl.kernel(
        _tr_body,
        out_type=jax.ShapeDtypeStruct((1000000 * D,), jnp.float32),
        mesh=plsc.VectorSubcoreMesh(core_axis_name="c", subcore_axis_name="s"),
        compiler_params=pltpu.CompilerParams(
            use_tc_tiling_on_sc=True, needs_layout_passes=False
        ),
        scratch_types=[
            pltpu.VMEM((NB2 * 4, 8, 128), jnp.float32),
        ]
        + [pltpu.VMEM((32 * 128,), jnp.float32)] * NB2
        + [pltpu.SemaphoreType.DMA]
        + [pltpu.SemaphoreType.DMA] * NB2,
    )(W.T, w2t)
    return wl.reshape(1000000, D)


def kernel(x, W):
    idx = (
        x.astype(jnp.int32)
        .reshape(32, 128, 25, 8)
        .transpose(2, 0, 3, 1)
        .reshape(NCHUNK, GROUP)
    )
    out5 = pl.kernel(
        _emb_body,
        out_type=jax.ShapeDtypeStruct((TT, 4, 32, 8 * GROUP), jnp.float32),
        mesh=plsc.VectorSubcoreMesh(core_axis_name="c", subcore_axis_name="s"),
        compiler_params=pltpu.CompilerParams(
            use_tc_tiling_on_sc=False, needs_layout_passes=False
        ),
        scratch_types=[
            pltpu.VMEM((GPW, GROUP), jnp.int32),
            pltpu.VMEM((NBUF, GROUP, D), jnp.float32),
            pltpu.VMEM((NBUF, D * GROUP), jnp.float32),
            pltpu.SemaphoreType.DMA,
        ]
        + [pltpu.SemaphoreType.DMA] * NBUF,
    )(idx, _relayout_table(W))
    return (
        out5.reshape(TT, 4, 32, 8, GROUP)
        .transpose(2, 4, 0, 1, 3)
        .reshape(BT, TT, D)
    )


# speedup vs baseline: 1.2733x; 1.2733x over previous
import jax
import jax.numpy as jnp
from jax import lax
from jax.experimental import pallas as pl
from jax.experimental.pallas import tpu as pltpu
from jax.experimental.pallas import tpu_sc as plsc

NC, NS = 2, 16
NW = NC * NS
BT = 4096
TT = 200
D = 32
GROUP = 128
NCHUNK = BT * TT // GROUP
GPW = NCHUNK // NW
NBUF = 4


def _emb_body(idx_hbm, w_hbm, out_hbm, idx_v, g_v, gt_v, gsem, *osems):
    wid = lax.axis_index("s") * NC + lax.axis_index("c")
    r0 = wid * GPW
    pltpu.sync_copy(idx_hbm.at[pl.ds(r0, GPW)], idx_v)
    iota = lax.iota(jnp.int32, 16)
    colidx = [(iota + k) & 15 for k in range(16)]
    sidx = [c * GROUP + iota for c in colidx]

    def fire_g(k, b):
        pltpu.async_copy(w_hbm.at[idx_v.at[k]], g_v.at[b], gsem)

    def drain_g(b):
        pltpu.make_async_copy(w_hbm.at[pl.ds(0, GROUP)], g_v.at[b], gsem).wait()

    def transpose(b):
        @plsc.parallel_loop(0, GROUP, step=16)
        def _blk(j0):
            rows = iota + j0
            for d0 in (0, 16):
                for k in range(16):
                    vals = plsc.load_gather(g_v.at[b], [rows, colidx[k] + d0])
                    plsc.store_scatter(
                        gt_v.at[b], [sidx[k] + (d0 * GROUP + j0)], vals
                    )

    def fire_out(k, b):
        r = r0 + k
        t = (r >> 8) * 8 + (r & 7)
        b_hi = (r >> 3) & 31
        for d_hi in range(4):
            pltpu.async_copy(
                gt_v.at[b].at[pl.ds(d_hi * 1024, 1024)],
                out_hbm.at[t, d_hi, b_hi],
                osems[b],
            )

    def drain_out(b):
        for d_hi in range(4):
            pltpu.make_async_copy(
                gt_v.at[b].at[pl.ds(d_hi * 1024, 1024)],
                out_hbm.at[0, d_hi, 0],
                osems[b],
            ).wait()

    for b in range(NBUF):
        fire_g(b, b)

    @pl.loop(0, GPW - NBUF, step=NBUF)
    def _steady(kk):
        for b in range(NBUF):
            k = kk + b
            drain_g(b)

            @pl.when(kk > 0)
            def _lagged_drain():
                drain_out(b)

            transpose(b)
            fire_out(k, b)
            fire_g(k + NBUF, b)

    for b in range(NBUF):
        k = GPW - NBUF + b
        drain_g(b)
        drain_out(b)
        transpose(b)
        fire_out(k, b)
    for b in range(NBUF):
        drain_out(b)


VB = 7812
BPW = 245
BIMAX = VB - 1
NB2 = 5
TAIL0 = VB * 128


def _tr_body(wt_hbm, w2t_hbm, wl_hbm, s3, *rest):
    gts = rest[:NB2]
    gsem = rest[NB2]
    osems = rest[NB2 + 1:]
    wid = lax.axis_index("s") * NC + lax.axis_index("c")
    base = wid * BPW
    iota = lax.iota(jnp.int32, 16)
    colidx = [(iota + k) & 15 for k in range(16)]
    sidxa = [c * D + iota for c in colidx]
    dhi = [iota >> 3, (iota >> 3) + 2]
    dlo = iota & 7

    def c0_of(n):
        return pl.multiple_of(jnp.minimum(base + n, BIMAX) * 128, 128)

    def fire_in(n, s):
        c0 = c0_of(n)
        for r in range(4):
            pltpu.async_copy(
                wt_hbm.at[pl.ds(8 * r, 8), pl.ds(c0, 128)],
                s3.at[s * 4 + r],
                gsem,
            )

    def drain_in(s):
        for r in range(4):
            pltpu.make_async_copy(
                wt_hbm.at[pl.ds(0, 8), pl.ds(0, 128)], s3.at[s * 4 + r], gsem
            ).wait()

    def transpose(s):
        @plsc.parallel_loop(0, 8)
        def _blk(j0i):
            j0 = j0i * 16
            for h in (0, 1):
                for k in range(16):
                    vals = plsc.load_gather(
                        s3.at[pl.ds(s * 4, 4)], [dhi[h], dlo, colidx[k] + j0]
                    )
                    plsc.store_scatter(
                        gts[s], [sidxa[k] + (j0 * D + 16 * h)], vals
                    )

    def fire_out(n, s):
        pltpu.async_copy(
            gts[s],
            wl_hbm.at[pl.ds(pl.multiple_of(c0_of(n) * D, 4096), 4096)],
            osems[s],
        )

    def drain_out(s):
        pltpu.make_async_copy(
            gts[s], wl_hbm.at[pl.ds(0, 4096)], osems[s]
        ).wait()

    for s in range(NB2):
        fire_in(s, s)

    @pl.loop(0, BPW - NB2, step=NB2)
    def _steady(nn):
        for s in range(NB2):
            n = nn + s
            drain_in(s)

            @pl.when(nn > 0)
            def _lagged_drain():
                drain_out(s)

            transpose(s)
            fire_out(n, s)
            fire_in(n + NB2, s)

    for s in range(NB2):
        n = BPW - NB2 + s
        drain_in(s)
        drain_out(s)
        transpose(s)
        fire_out(n, s)
    for s in range(NB2):
        drain_out(s)

    @pl.when(wid == NW - 1)
    def _tail():
        for r in range(4):
            pltpu.sync_copy(w2t_hbm.at[pl.ds(8 * r, 8)], s3.at[r])
        transpose(0)
        pltpu.sync_copy(
            gts[0].at[pl.ds(0, 64 * D)], wl_hbm.at[pl.ds(TAIL0 * D, 64 * D)]
        )


def _relayout_table(W):
    w2t = jnp.pad(W[TAIL0:], ((0, 64), (0, 0))).T
    wl = pl.kernel(
        _tr_body,
        out_type=jax.ShapeDtypeStruct((1000000 * D,), jnp.float32),
        mesh=plsc.VectorSubcoreMesh(core_axis_name="c", subcore_axis_name="s"),
        compiler_params=pltpu.CompilerParams(
            use_tc_tiling_on_sc=True, needs_layout_passes=False
        ),
        scratch_types=[
            pltpu.VMEM((NB2 * 4, 8, 128), jnp.float32),
        ]
        + [pltpu.VMEM((32 * 128,), jnp.float32)] * NB2
        + [pltpu.SemaphoreType.DMA]
        + [pltpu.SemaphoreType.DMA] * NB2,
    )(W.T, w2t)
    return wl.reshape(1000000, D)


def kernel(x, W):
    idx = (
        x.astype(jnp.int32)
        .reshape(32, 128, 25, 8)
        .transpose(2, 0, 3, 1)
        .reshape(NCHUNK, GROUP)
    )
    out5 = pl.kernel(
        _emb_body,
        out_type=jax.ShapeDtypeStruct((TT, 4, 32, 8 * GROUP), jnp.float32),
        mesh=plsc.VectorSubcoreMesh(core_axis_name="c", subcore_axis_name="s"),
        compiler_params=pltpu.CompilerParams(
            use_tc_tiling_on_sc=False, needs_layout_passes=False
        ),
        scratch_types=[
            pltpu.VMEM((GPW, GROUP), jnp.int32),
            pltpu.VMEM((NBUF, GROUP, D), jnp.float32),
            pltpu.VMEM((NBUF, D * GROUP), jnp.float32),
            pltpu.SemaphoreType.DMA,
        ]
        + [pltpu.SemaphoreType.DMA] * NBUF,
    )(idx, _relayout_table(W))
    return (
        out5.reshape(TT, 4, 32, 8, GROUP)
        .transpose(2, 4, 0, 1, 3)
        .reshape(BT, TT, D)
    )
